# Initial kernel scaffold; baseline (speedup 1.0000x reference)
import jax
import jax.numpy as jnp
from jax.experimental import pallas as pl


def kernel(x_query, W, prompt_memory, prompt_keys):
    raise NotImplementedError("write your pallas kernel here")


# speedup vs baseline: 1.6263x; 1.6263x over previous
import functools

import jax
import jax.numpy as jnp
from jax import lax
from jax.experimental import pallas as pl
from jax.experimental.pallas import tpu as pltpu
from jax.experimental.pallas import tpu_sc as plsc

_B = 1024
_E = 1024
_KD = 256
_M = 8192
_L = 8
_K = 8
_RB = 128
_NW = 32
_QPW = _B // _NW


def _topk_body(x_ref, w_ref, keys_ref, idx_ref, wts_ref, kn_ref):
    @pl.when(pl.program_id(0) == 0)
    def _():
        k = keys_ref[...]
        ks = jnp.sum(k * k, axis=1, keepdims=True)
        kn_ref[...] = k / jnp.maximum(jnp.sqrt(ks), 1e-12)

    x = x_ref[...]
    xw = lax.dot_general(x, w_ref[...], (((1,), (1,)), ((), ())),
                         preferred_element_type=jnp.float32)
    xs = jnp.sum(xw * xw, axis=1, keepdims=True)
    xn = xw / jnp.maximum(jnp.sqrt(xs), 1e-12)
    sim = lax.dot_general(xn, kn_ref[...], (((1,), (1,)), ((), ())),
                          preferred_element_type=jnp.float32)

    iota = lax.broadcasted_iota(jnp.int32, sim.shape, 1)
    scores = sim
    vals = []
    idxs = []
    for _ in range(_K):
        mx = jnp.max(scores, axis=1, keepdims=True)
        ix = jnp.min(jnp.where(scores == mx, iota, jnp.int32(_M)),
                     axis=1, keepdims=True)
        vals.append(mx)
        idxs.append(ix)
        scores = jnp.where(iota == ix, -jnp.inf, scores)
    tv = jnp.concatenate(vals, axis=1)
    ti = jnp.concatenate(idxs, axis=1)
    e = jnp.exp(tv - tv[:, :1])
    w = e / jnp.sum(e, axis=1, keepdims=True)
    idx_ref[...] = ti
    wts_ref[...] = w


def _topk_call(x_query, W, prompt_keys):
    return pl.pallas_call(
        _topk_body,
        grid=(_B // _RB,),
        in_specs=[
            pl.BlockSpec((_RB, _E), lambda i: (i, 0)),
            pl.BlockSpec((_KD, _E), lambda i: (0, 0)),
            pl.BlockSpec((_M, _KD), lambda i: (0, 0)),
        ],
        out_specs=[
            pl.BlockSpec((_RB, _K), lambda i: (i, 0)),
            pl.BlockSpec((_RB, _K), lambda i: (i, 0)),
        ],
        out_shape=[
            jax.ShapeDtypeStruct((_B, _K), jnp.int32),
            jax.ShapeDtypeStruct((_B, _K), jnp.float32),
        ],
        scratch_shapes=[pltpu.VMEM((_M, _KD), jnp.float32)],
    )(x_query, W, prompt_keys)


def _sc_combine_body(mem_ref, idx_ref, wts_ref, out_ref,
                     idx_v, w_v, rows_v, acc_v, sem):
    wid = lax.axis_index("s") * 2 + lax.axis_index("c")
    base = wid * _QPW
    pltpu.sync_copy(idx_ref.at[pl.ds(base * _K, _QPW * _K)], idx_v)
    pltpu.sync_copy(wts_ref.at[pl.ds(base * 16, _QPW * 16)], w_v)

    def per_query(q, carry):
        pltpu.async_copy(mem_ref.at[idx_v.at[pl.ds(q * _K, _K)]],
                         rows_v, sem).wait()
        wv = w_v[pl.ds(q * 16, 16)]
        ws = [wv[j] for j in range(_K)]

        def chunk(c, carry2):
            off = c * 16
            a = ws[0] * rows_v[0, pl.ds(off, 16)]
            for j in range(1, _K):
                a = a + ws[j] * rows_v[j, pl.ds(off, 16)]
            acc_v[pl.ds(off, 16)] = a
            return carry2

        lax.fori_loop(0, (_L * _E) // 16, chunk, 0)
        pltpu.sync_copy(acc_v, out_ref.at[base + q])
        return carry

    lax.fori_loop(0, _QPW, per_query, 0)


_sc_combine = functools.partial(
    pl.kernel,
    out_type=jax.ShapeDtypeStruct((_B, _L * _E), jnp.float32),
    mesh=plsc.VectorSubcoreMesh(core_axis_name="c", subcore_axis_name="s"),
    scratch_types=[
        pltpu.VMEM((_QPW * _K,), jnp.int32),
        pltpu.VMEM((_QPW * 16,), jnp.float32),
        pltpu.VMEM((_K, _L * _E), jnp.float32),
        pltpu.VMEM((_L * _E,), jnp.float32),
        pltpu.SemaphoreType.DMA,
    ],
)(_sc_combine_body)


def kernel(x_query, W, prompt_memory, prompt_keys):
    idx, wts = _topk_call(x_query, W, prompt_keys)
    mem_flat = prompt_memory.reshape(_M, _L * _E)
    idx_flat = idx.reshape(_B * _K)
    wts_pad = jnp.pad(wts, ((0, 0), (0, 16 - _K))).reshape(_B * 16)
    out = _sc_combine(mem_flat, idx_flat, wts_pad)
    return out.reshape(_B, _L, _E)
